# baseline (speedup 1.0000x reference)
import functools

import jax
import jax.numpy as jnp
from jax import lax
from jax.experimental import pallas as pl
from jax.experimental.pallas import tpu as pltpu
from jax.experimental.pallas import tpu_sc as plsc

_N = 10000
_D = 128
_H1 = 8
_C1 = 8
_C2 = 64
_NP = 10112
_ROW = 80
_ADW = 16
_B = 128
_NW = 32
_NBATCH = 82
_EP = _NW * _NBATCH * _B
_RPS = _NP // 16


def _lrelu(x):
    return jnp.maximum(x, 0.2 * x)


def _stage_a_body(x_ref, w1_ref, as1_ref, ad1_ref, t1_ref, adt_ref, mub_ref):
    xp = jnp.dot(x_ref[...], w1_ref[...], preferred_element_type=jnp.float32)
    als = jnp.dot(xp, as1_ref[...], preferred_element_type=jnp.float32)
    ald = jnp.dot(xp, ad1_ref[...], preferred_element_type=jnp.float32)
    mub = _lrelu(jnp.max(als, axis=0) + jnp.max(ald, axis=0))
    t1 = jnp.concatenate(
        [xp, als, jnp.zeros((_N, _ROW - _H1 * _C1 - _H1), jnp.float32)], axis=1)
    t1_ref[...] = jnp.concatenate(
        [t1, jnp.zeros((_NP - _N, _ROW), jnp.float32)], axis=0)
    adt = jnp.concatenate(
        [ald, jnp.zeros((_N, _ADW - _H1), jnp.float32)], axis=1)
    adt_ref[...] = jnp.concatenate(
        [adt, jnp.zeros((_NP - _N, _ADW), jnp.float32)], axis=0)
    mub_ref[...] = jnp.concatenate(
        [mub, jnp.full((16 - _H1,), 1e9, jnp.float32)]).reshape(1, 16)


_stage_a = pl.pallas_call(
    _stage_a_body,
    out_shape=(jax.ShapeDtypeStruct((_NP, _ROW), jnp.float32),
               jax.ShapeDtypeStruct((_NP, _ADW), jnp.float32),
               jax.ShapeDtypeStruct((1, 16), jnp.float32)),
)


def _stage_c_body(pa_ref, pb_ref, b1_ref, w2_ref, as2_ref, ad2_ref,
                  t2_ref, adt_ref, mub_ref):
    acc = pa_ref[...] + pb_ref[...]
    num = acc[:, 0:_H1 * _C1]
    wsum = acc[:, _H1 * _C1:_H1 * _C1 + _H1]
    ksel = (lax.broadcasted_iota(jnp.int32, (_H1, _H1 * _C1), 0)
            == lax.broadcasted_iota(jnp.int32, (_H1, _H1 * _C1), 1) // _C1
            ).astype(jnp.float32)
    recip = 1.0 / jnp.maximum(wsum, 1e-30)
    h1 = jnp.maximum(
        num * jnp.dot(recip, ksel, preferred_element_type=jnp.float32)
        + b1_ref[...][None, :], 0.0)
    xp2 = jnp.dot(h1, w2_ref[...], preferred_element_type=jnp.float32)
    as2 = lax.dot_general(xp2, as2_ref[...], (((1,), (1,)), ((), ())),
                          preferred_element_type=jnp.float32)
    ad2 = lax.dot_general(xp2, ad2_ref[...], (((1,), (1,)), ((), ())),
                          preferred_element_type=jnp.float32)
    mub2 = _lrelu(jnp.max(as2) + jnp.max(ad2))
    t2_ref[...] = jnp.concatenate(
        [xp2, as2, jnp.zeros((_NP, _ROW - _C2 - 1), jnp.float32)], axis=1)
    adt_ref[...] = jnp.concatenate(
        [ad2, jnp.zeros((_NP, _ADW - 1), jnp.float32)], axis=1)
    mub_ref[...] = jnp.concatenate(
        [mub2.reshape(1), jnp.full((15,), 1e9, jnp.float32)]).reshape(1, 16)


_stage_c = pl.pallas_call(
    _stage_c_body,
    out_shape=(jax.ShapeDtypeStruct((_NP, _ROW), jnp.float32),
               jax.ShapeDtypeStruct((_NP, _ADW), jnp.float32),
               jax.ShapeDtypeStruct((1, 16), jnp.float32)),
)


def _stage_d_body(pa_ref, pb_ref, b2_ref, fcw_ref, fcb_ref, out_ref):
    acc = pa_ref[...] + pb_ref[...]
    num = acc[0:_N, 0:_C2]
    wsum = acc[0:_N, _C2:_C2 + 1]
    h2 = num * (1.0 / jnp.maximum(wsum, 1e-30)) + b2_ref[...][None, :]
    out_ref[...] = (jnp.dot(h2, fcw_ref[...], preferred_element_type=jnp.float32)
                    + fcb_ref[...][None, :])


_stage_d = pl.pallas_call(
    _stage_d_body,
    out_shape=jax.ShapeDtypeStruct((_N, 2), jnp.float32),
)


def _make_edge_kernel(multi_head: bool):
    mesh = plsc.VectorSubcoreMesh(core_axis_name="c", subcore_axis_name="s")

    @functools.partial(
        pl.kernel,
        out_type=jax.ShapeDtypeStruct((2, _NP, _ROW), jnp.float32),
        mesh=mesh,
        compiler_params=pltpu.CompilerParams(use_tc_tiling_on_sc=False),
        scratch_types=[
            pltpu.VMEM((_B, _ROW), jnp.float32),
            pltpu.VMEM((_B, _ROW), jnp.float32),
            pltpu.VMEM((_B, _ADW), jnp.float32),
            pltpu.VMEM((_B, _ADW), jnp.float32),
            pltpu.VMEM((_B, _ROW), jnp.float32),
            pltpu.VMEM((_B, _ROW), jnp.float32),
            pltpu.VMEM((_NBATCH, _B), jnp.int32),
            pltpu.VMEM((_NBATCH, _B), jnp.int32),
            pltpu.VMEM((16,), jnp.float32),
            pltpu.VMEM_SHARED((_NP, _ROW), jnp.float32),
            pltpu.VMEM_SHARED((_NP, _ADW), jnp.float32),
            pltpu.SemaphoreType.DMA,
            pltpu.SemaphoreType.DMA,
            pltpu.SemaphoreType.DMA,
            pltpu.SemaphoreType.DMA,
            pltpu.SemaphoreType.DMA,
            pltpu.SemaphoreType.DMA,
        ],
    )
    def edge_kernel(t_hbm, ad_hbm, mub_hbm, src_hbm, dst_hbm, out_hbm,
                    rows0_v, rows1_v, adr0_v, adr1_v, msg0_v, msg1_v,
                    srcb_v, dstb_v, mub_v, acc_sh, ad_sh,
                    gt0, gt1, ga0, ga1, sc0, sc1):
        c = lax.axis_index("c")
        s = lax.axis_index("s")
        wid = s * 2 + c
        pltpu.sync_copy(src_hbm.at[wid], srcb_v)
        pltpu.sync_copy(dst_hbm.at[wid], dstb_v)
        pltpu.sync_copy(mub_hbm, mub_v)
        stage0 = s * _RPS
        pltpu.sync_copy(ad_hbm.at[pl.ds(stage0, _RPS)],
                        ad_sh.at[pl.ds(stage0, _RPS)])

        rows = (rows0_v, rows1_v)
        adr = (adr0_v, adr1_v)
        msg = (msg0_v, msg1_v)
        gt = (gt0, gt1)
        ga = (ga0, ga1)
        sc = (sc0, sc1)

        def _zrow(e, carry):
            for j in range(_ROW // 16):
                msg0_v[e, pl.ds(16 * j, 16)] = jnp.zeros((16,), jnp.float32)
            return carry
        lax.fori_loop(0, _B, _zrow, 0)
        base_row = s * _RPS
        nfull = _RPS // _B
        for k in range(nfull):
            pltpu.sync_copy(msg0_v, acc_sh.at[pl.ds(base_row + k * _B, _B)])
        rem = _RPS - nfull * _B
        pltpu.sync_copy(msg0_v.at[pl.ds(0, rem)],
                        acc_sh.at[pl.ds(base_row + nfull * _B, rem)])
        plsc.subcore_barrier()

        def _start_gather(b, i):
            pltpu.async_copy(t_hbm.at[srcb_v.at[b]], rows[i], gt[i])
            pltpu.async_copy(ad_sh.at[dstb_v.at[b]], adr[i], ga[i])

        def _wait_gather(b, i):
            pltpu.make_async_copy(t_hbm.at[srcb_v.at[b]], rows[i], gt[i]).wait()
            pltpu.make_async_copy(ad_sh.at[dstb_v.at[b]], adr[i], ga[i]).wait()

        def _wait_scatter(b, i):
            pltpu.make_async_copy(msg[i], acc_sh.at[dstb_v.at[b]], sc[i]).wait()

        def _compute(i):
            rows_v, adr_v, msg_v = rows[i], adr[i], msg[i]
            mub_vec = mub_v[...]
            iot = lax.iota(jnp.int32, 16)

            @plsc.parallel_loop(0, _B, unroll=4)
            def _edge(e):
                asv = rows_v[e, pl.ds(64, 16)]
                adv = adr_v[e, pl.ds(0, 16)]
                al = asv + adv
                al = jnp.maximum(al, 0.2 * al)
                w = jnp.exp(al - mub_vec)
                msg_v[e, pl.ds(64, 16)] = w
                for j in range(4):
                    if multi_head:
                        wj = jnp.where(iot < 8, w[2 * j], w[2 * j + 1])
                    else:
                        wj = jnp.broadcast_to(w[0], (16,))
                    msg_v[e, pl.ds(16 * j, 16)] = (
                        rows_v[e, pl.ds(16 * j, 16)] * wj)

        _start_gather(0, 0)

        def _pair(k, carry):
            for i in range(2):
                b = 2 * k + i
                _wait_gather(b, i)
                if i == 0:
                    _start_gather(b + 1, 1)
                else:
                    @pl.when(k < _NBATCH // 2 - 1)
                    def _():
                        _start_gather(b + 1, 0)

                @pl.when(k > 0)
                def _():
                    _wait_scatter(b, i)
                _compute(i)
                pltpu.async_copy(msg[i], acc_sh.at[dstb_v.at[b]], sc[i],
                                 add=True)
            return carry
        lax.fori_loop(0, _NBATCH // 2, _pair, 0)
        _wait_scatter(_NBATCH - 2, 0)
        _wait_scatter(_NBATCH - 1, 1)
        plsc.subcore_barrier()

        for k in range(nfull):
            r0 = base_row + k * _B
            pltpu.sync_copy(acc_sh.at[pl.ds(r0, _B)], rows0_v)
            pltpu.sync_copy(rows0_v, out_hbm.at[c, pl.ds(r0, _B)])
        r0 = base_row + nfull * _B
        pltpu.sync_copy(acc_sh.at[pl.ds(r0, rem)], rows0_v.at[pl.ds(0, rem)])
        pltpu.sync_copy(rows0_v.at[pl.ds(0, rem)],
                        out_hbm.at[c, pl.ds(r0, rem)])

    return edge_kernel


_edge_l1 = _make_edge_kernel(multi_head=True)
_edge_l2 = _make_edge_kernel(multi_head=False)


def kernel(x, edge_index, W1, a_src1, a_dst1, b1, W2, a_src2, a_dst2, b2,
           fc_W, fc_b):
    loop = jnp.arange(_N, dtype=jnp.int32)
    padn = _EP - (320000 + _N)
    pad_dst = (_N + 1 + (jnp.arange(padn, dtype=jnp.int32)
                         % (_NP - _N - 1))).astype(jnp.int32)
    src = jnp.concatenate([edge_index[0], loop,
                           jnp.zeros((padn,), jnp.int32)])
    dst = jnp.concatenate([edge_index[1], loop, pad_dst])
    src = src.reshape(_NW, _NBATCH, _B)
    dst = dst.reshape(_NW, _NBATCH, _B)

    hsel = (jnp.arange(_H1 * _C1)[:, None] // _C1 == jnp.arange(_H1)[None, :])
    as1m = a_src1.reshape(_H1 * _C1, 1) * hsel
    ad1m = a_dst1.reshape(_H1 * _C1, 1) * hsel
    t1, ad1t, mub1 = _stage_a(x, W1, as1m, ad1m)
    acc1 = _edge_l1(t1, ad1t, mub1.reshape(16), src, dst)
    t2, ad2t, mub2 = _stage_c(acc1[0], acc1[1], b1, W2, a_src2, a_dst2)
    acc2 = _edge_l2(t2, ad2t, mub2.reshape(16), src, dst)
    return _stage_d(acc2[0], acc2[1], b2, fc_W, fc_b)


# speedup vs baseline: 2.0634x; 2.0634x over previous
import functools

import jax
import jax.numpy as jnp
from jax import lax
from jax.experimental import pallas as pl
from jax.experimental.pallas import tpu as pltpu
from jax.experimental.pallas import tpu_sc as plsc

_N = 10000
_D = 128
_H1 = 8
_C1 = 8
_C2 = 64
_NP = 10112
_ROW = 80
_ADW = 16
_B = 128
_NW = 32
_NBATCH = 82
_EP = _NW * _NBATCH * _B
_RPS = _NP // 16


def _lrelu(x):
    return jnp.maximum(x, 0.2 * x)


def _stage_a_body(x_ref, w1_ref, as1_ref, ad1_ref, t1_ref, adt_ref, mub_ref):
    xp = jnp.dot(x_ref[...], w1_ref[...], preferred_element_type=jnp.float32)
    als = jnp.dot(xp, as1_ref[...], preferred_element_type=jnp.float32)
    ald = jnp.dot(xp, ad1_ref[...], preferred_element_type=jnp.float32)
    mub = _lrelu(jnp.max(als, axis=0) + jnp.max(ald, axis=0))
    t1 = jnp.concatenate(
        [xp, als, jnp.zeros((_N, _ROW - _H1 * _C1 - _H1), jnp.float32)], axis=1)
    poison = jnp.concatenate(
        [jnp.zeros((_NP - _N, _H1 * _C1), jnp.float32),
         jnp.full((_NP - _N, _ROW - _H1 * _C1), -1000.0, jnp.float32)], axis=1)
    t1_ref[...] = jnp.concatenate([t1, poison], axis=0)
    adt = jnp.concatenate(
        [ald, jnp.zeros((_N, _ADW - _H1), jnp.float32)], axis=1)
    adt_ref[...] = jnp.concatenate(
        [adt, jnp.zeros((_NP - _N, _ADW), jnp.float32)], axis=0)
    mub_ref[...] = jnp.concatenate(
        [mub, jnp.full((16 - _H1,), 1e9, jnp.float32)]).reshape(1, 16)


_stage_a = pl.pallas_call(
    _stage_a_body,
    out_shape=(jax.ShapeDtypeStruct((_NP, _ROW), jnp.float32),
               jax.ShapeDtypeStruct((_NP, _ADW), jnp.float32),
               jax.ShapeDtypeStruct((1, 16), jnp.float32)),
)


def _stage_c_body(pa_ref, pb_ref, b1_ref, w2_ref, as2_ref, ad2_ref,
                  t2_ref, adt_ref, mub_ref):
    acc = pa_ref[...] + pb_ref[...]
    num = acc[:, 0:_H1 * _C1]
    wsum = acc[:, _H1 * _C1:_H1 * _C1 + _H1]
    ksel = (lax.broadcasted_iota(jnp.int32, (_H1, _H1 * _C1), 0)
            == lax.broadcasted_iota(jnp.int32, (_H1, _H1 * _C1), 1) // _C1
            ).astype(jnp.float32)
    recip = 1.0 / jnp.maximum(wsum, 1e-30)
    h1 = jnp.maximum(
        num * jnp.dot(recip, ksel, preferred_element_type=jnp.float32)
        + b1_ref[...][None, :], 0.0)
    xp2 = jnp.dot(h1, w2_ref[...], preferred_element_type=jnp.float32)
    as2 = lax.dot_general(xp2, as2_ref[...], (((1,), (1,)), ((), ())),
                          preferred_element_type=jnp.float32)
    ad2 = lax.dot_general(xp2, ad2_ref[...], (((1,), (1,)), ((), ())),
                          preferred_element_type=jnp.float32)
    mub2 = _lrelu(jnp.max(as2[0:_N]) + jnp.max(ad2[0:_N]))
    as2m = jnp.where(
        lax.broadcasted_iota(jnp.int32, (_NP, 1), 0) < _N, as2, -1000.0)
    t2_ref[...] = jnp.concatenate(
        [xp2, as2m, jnp.zeros((_NP, _ROW - _C2 - 1), jnp.float32)], axis=1)
    adt_ref[...] = jnp.concatenate(
        [ad2, jnp.zeros((_NP, _ADW - 1), jnp.float32)], axis=1)
    mub_ref[...] = jnp.concatenate(
        [mub2.reshape(1), jnp.full((15,), 1e9, jnp.float32)]).reshape(1, 16)


_stage_c = pl.pallas_call(
    _stage_c_body,
    out_shape=(jax.ShapeDtypeStruct((_NP, _ROW), jnp.float32),
               jax.ShapeDtypeStruct((_NP, _ADW), jnp.float32),
               jax.ShapeDtypeStruct((1, 16), jnp.float32)),
)


def _stage_d_body(pa_ref, pb_ref, b2_ref, fcw_ref, fcb_ref, out_ref):
    acc = pa_ref[...] + pb_ref[...]
    num = acc[0:_N, 0:_C2]
    wsum = acc[0:_N, _C2:_C2 + 1]
    h2 = num * (1.0 / jnp.maximum(wsum, 1e-30)) + b2_ref[...][None, :]
    out_ref[...] = (jnp.dot(h2, fcw_ref[...], preferred_element_type=jnp.float32)
                    + fcb_ref[...][None, :])


_stage_d = pl.pallas_call(
    _stage_d_body,
    out_shape=jax.ShapeDtypeStruct((_N, 2), jnp.float32),
)


def _make_edge_kernel(multi_head: bool):
    mesh = plsc.VectorSubcoreMesh(core_axis_name="c", subcore_axis_name="s")

    @functools.partial(
        pl.kernel,
        out_type=jax.ShapeDtypeStruct((2, _NP, _ROW), jnp.float32),
        mesh=mesh,
        compiler_params=pltpu.CompilerParams(use_tc_tiling_on_sc=False),
        scratch_types=[
            pltpu.VMEM((_B, _ROW), jnp.float32),
            pltpu.VMEM((_B, _ROW), jnp.float32),
            pltpu.VMEM((_B, _ADW), jnp.float32),
            pltpu.VMEM((_B, _ADW), jnp.float32),
            pltpu.VMEM((_B, _ROW), jnp.float32),
            pltpu.VMEM((_B, _ROW), jnp.float32),
            pltpu.VMEM((_NBATCH, _B), jnp.int32),
            pltpu.VMEM((_NBATCH, _B), jnp.int32),
            pltpu.VMEM((16,), jnp.float32),
            pltpu.VMEM_SHARED((_NP, _ROW), jnp.float32),
            pltpu.VMEM_SHARED((_NP, _ADW), jnp.float32),
            pltpu.SemaphoreType.DMA,
            pltpu.SemaphoreType.DMA,
            pltpu.SemaphoreType.DMA,
            pltpu.SemaphoreType.DMA,
            pltpu.SemaphoreType.DMA,
            pltpu.SemaphoreType.DMA,
        ],
    )
    def edge_kernel(t_hbm, ad_hbm, mub_hbm, src_hbm, dst_hbm, out_hbm,
                    rows0_v, rows1_v, adr0_v, adr1_v, msg0_v, msg1_v,
                    srcb_v, dstb_v, mub_v, acc_sh, ad_sh,
                    gt0, gt1, ga0, ga1, sc0, sc1):
        c = lax.axis_index("c")
        s = lax.axis_index("s")
        wid = s * 2 + c
        pltpu.sync_copy(src_hbm.at[wid], srcb_v)
        pltpu.sync_copy(dst_hbm.at[wid], dstb_v)
        pltpu.sync_copy(mub_hbm, mub_v)
        stage0 = s * _RPS
        pltpu.sync_copy(ad_hbm.at[pl.ds(stage0, _RPS)],
                        ad_sh.at[pl.ds(stage0, _RPS)])

        rows = (rows0_v, rows1_v)
        adr = (adr0_v, adr1_v)
        msg = (msg0_v, msg1_v)
        gt = (gt0, gt1)
        ga = (ga0, ga1)
        sc = (sc0, sc1)

        def _zrow(e, carry):
            for j in range(_ROW // 16):
                msg0_v[e, pl.ds(16 * j, 16)] = jnp.zeros((16,), jnp.float32)
            return carry
        lax.fori_loop(0, _B, _zrow, 0)
        base_row = s * _RPS
        nfull = _RPS // _B
        for k in range(nfull):
            pltpu.sync_copy(msg0_v, acc_sh.at[pl.ds(base_row + k * _B, _B)])
        rem = _RPS - nfull * _B
        pltpu.sync_copy(msg0_v.at[pl.ds(0, rem)],
                        acc_sh.at[pl.ds(base_row + nfull * _B, rem)])
        plsc.subcore_barrier()

        def _start_gather(b, i):
            pltpu.async_copy(t_hbm.at[srcb_v.at[b]], rows[i], gt[i])
            pltpu.async_copy(ad_sh.at[dstb_v.at[b]], adr[i], ga[i])

        def _wait_gather(b, i):
            pltpu.make_async_copy(t_hbm.at[srcb_v.at[b]], rows[i], gt[i]).wait()
            pltpu.make_async_copy(ad_sh.at[dstb_v.at[b]], adr[i], ga[i]).wait()

        def _wait_scatter(b, i):
            pltpu.make_async_copy(msg[i], acc_sh.at[dstb_v.at[b]], sc[i]).wait()

        def _compute(i):
            rows_v, adr_v, msg_v = rows[i], adr[i], msg[i]
            mub_vec = mub_v[...]
            iot = lax.iota(jnp.int32, 16)

            @plsc.parallel_loop(0, _B, unroll=4)
            def _edge(e):
                asv = rows_v[e, pl.ds(64, 16)]
                adv = adr_v[e, pl.ds(0, 16)]
                al = asv + adv
                al = jnp.maximum(al, 0.2 * al)
                w = jnp.exp(al - mub_vec)
                msg_v[e, pl.ds(64, 16)] = w
                for j in range(4):
                    if multi_head:
                        wj = jnp.where(iot < 8, w[2 * j], w[2 * j + 1])
                    else:
                        wj = jnp.broadcast_to(w[0], (16,))
                    msg_v[e, pl.ds(16 * j, 16)] = (
                        rows_v[e, pl.ds(16 * j, 16)] * wj)

        _start_gather(0, 0)

        def _pair(k, carry):
            for i in range(2):
                b = 2 * k + i
                _wait_gather(b, i)
                if i == 0:
                    _start_gather(b + 1, 1)
                else:
                    @pl.when(k < _NBATCH // 2 - 1)
                    def _():
                        _start_gather(b + 1, 0)

                @pl.when(k > 0)
                def _():
                    _wait_scatter(b, i)
                _compute(i)
                pltpu.async_copy(msg[i], acc_sh.at[dstb_v.at[b]], sc[i],
                                 add=True)
            return carry
        lax.fori_loop(0, _NBATCH // 2, _pair, 0)
        _wait_scatter(_NBATCH - 2, 0)
        _wait_scatter(_NBATCH - 1, 1)
        plsc.subcore_barrier()

        for k in range(nfull):
            r0 = base_row + k * _B
            pltpu.sync_copy(acc_sh.at[pl.ds(r0, _B)], rows0_v)
            pltpu.sync_copy(rows0_v, out_hbm.at[c, pl.ds(r0, _B)])
        r0 = base_row + nfull * _B
        pltpu.sync_copy(acc_sh.at[pl.ds(r0, rem)], rows0_v.at[pl.ds(0, rem)])
        pltpu.sync_copy(rows0_v.at[pl.ds(0, rem)],
                        out_hbm.at[c, pl.ds(r0, rem)])

    return edge_kernel


_edge_l1 = _make_edge_kernel(multi_head=True)
_edge_l2 = _make_edge_kernel(multi_head=False)


def kernel(x, edge_index, W1, a_src1, a_dst1, b1, W2, a_src2, a_dst2, b2,
           fc_W, fc_b):
    loop = jnp.arange(_N, dtype=jnp.int32)
    padn = _EP - (320000 + _N)
    ar = jnp.arange(padn, dtype=jnp.int32)
    pad_src = _N + (ar % (_NP - _N))
    pad_dst = (ar * 89) % _N
    src = jnp.concatenate([edge_index[0], loop, pad_src])
    dst = jnp.concatenate([edge_index[1], loop, pad_dst])
    src = src.reshape(_NW, _NBATCH, _B)
    dst = dst.reshape(_NW, _NBATCH, _B)

    hsel = (jnp.arange(_H1 * _C1)[:, None] // _C1 == jnp.arange(_H1)[None, :])
    as1m = a_src1.reshape(_H1 * _C1, 1) * hsel
    ad1m = a_dst1.reshape(_H1 * _C1, 1) * hsel
    t1, ad1t, mub1 = _stage_a(x, W1, as1m, ad1m)
    acc1 = _edge_l1(t1, ad1t, mub1.reshape(16), src, dst)
    t2, ad2t, mub2 = _stage_c(acc1[0], acc1[1], b1, W2, a_src2, a_dst2)
    acc2 = _edge_l2(t2, ad2t, mub2.reshape(16), src, dst)
    return _stage_d(acc2[0], acc2[1], b2, fc_W, fc_b)
